# baseline (speedup 1.0000x reference)
import functools

import jax
import jax.numpy as jnp
from jax import lax
from jax.experimental import pallas as pl
from jax.experimental.pallas import tpu as pltpu
from jax.experimental.pallas import tpu_sc as plsc

N = 10000
E = 320000
D = 128
H = 128

NC = 2
NS = 16
L = 16
NW = NC * NS
EPT = E // NW
CHUNK = 80
NCHUNK = EPT // CHUNK
STRIPE = 624
TAIL = N - STRIPE * NS
TAIL_OFF = STRIPE * NS

CC = 128
RR = 80
NP = RR * CC

ECH = 128
CHT = E // ECH
NBIG = 17
PHCH = 40

BN = 1000
GRID = N // BN

_mesh = plsc.VectorSubcoreMesh(core_axis_name="c", subcore_axis_name="s")
_sc_params = pltpu.CompilerParams(needs_layout_passes=False)


def _zero_2d(ref):
    zeros = jnp.zeros((L,), jnp.float32)

    def zbody(i, carry):
        ref[i >> 3, pl.ds((i & 7) * L, L)] = zeros
        return carry

    lax.fori_loop(0, RR * CC // L, zbody, 0)


def _rowadd_to_shared(vref, shref):
    for k in range(RR // L):
        rid = lax.iota(jnp.int32, L) + (k * L)
        pltpu.sync_copy(vref.at[pl.ds(k * L, L)], shref.at[rid], add=True)


def _readout_shared(shref, out_hbm, c, s):
    @pl.when(s < RR // 8)
    def _():
        pltpu.sync_copy(shref.at[pl.ds(s * 8, 8)],
                        out_hbm.at[pl.ds(c * RR + s * 8, 8)])


@functools.partial(
    pl.kernel,
    out_type=jax.ShapeDtypeStruct((NC * RR, CC), jnp.float32),
    mesh=_mesh,
    compiler_params=_sc_params,
    scratch_types=[
        pltpu.VMEM_SHARED((RR, CC), jnp.float32),
        pltpu.VMEM((RR, CC), jnp.float32),
        pltpu.VMEM((EPT,), jnp.int32),
    ],
)
def _deg_kernel(dst_hbm, deg_out, deg_sh, deg_v, dstbuf):
    c = lax.axis_index("c")
    s = lax.axis_index("s")
    wid = s * NC + c
    pltpu.sync_copy(dst_hbm.at[pl.ds(wid * EPT, EPT)], dstbuf)
    _zero_2d(deg_v)
    pltpu.sync_copy(deg_v.at[pl.ds(0, RR // NS)],
                    deg_sh.at[pl.ds(s * (RR // NS), RR // NS)])
    ones = jnp.ones((L,), jnp.float32)

    def body(i, carry):
        idx = dstbuf[pl.ds(i * L, L)]
        plsc.addupdate_scatter(deg_v, [idx >> 7, idx & 127], ones)
        return carry

    lax.fori_loop(0, EPT // L, body, 0)
    plsc.subcore_barrier()
    _rowadd_to_shared(deg_v, deg_sh)
    plsc.subcore_barrier()
    _readout_shared(deg_sh, deg_out, c, s)


def _mm_body(x_ref, w1_ref, y_ref):
    y_ref[...] = jnp.dot(x_ref[...], w1_ref[...],
                         preferred_element_type=jnp.float32)


_mm_call = pl.pallas_call(
    _mm_body,
    grid=(GRID,),
    in_specs=[
        pl.BlockSpec((BN, D), lambda i: (i, 0)),
        pl.BlockSpec((D, H), lambda i: (0, 0)),
    ],
    out_specs=pl.BlockSpec((BN, H), lambda i: (i, 0)),
    out_shape=jax.ShapeDtypeStruct((N, H), jnp.float32),
)


def _scale_body(degt_ref, y_ref, norm_ref, yhat_ref):
    deg = jnp.sum(degt_ref[...], axis=1) + 1.0
    nrm = lax.rsqrt(deg)
    norm_ref[...] = nrm[:, None]
    yhat_ref[...] = y_ref[...] * nrm[:, None]


_scale_call = pl.pallas_call(
    _scale_body,
    grid=(GRID,),
    in_specs=[
        pl.BlockSpec((BN, NC), lambda i: (i, 0)),
        pl.BlockSpec((BN, H), lambda i: (i, 0)),
    ],
    out_specs=[
        pl.BlockSpec((BN, 1), lambda i: (i, 0)),
        pl.BlockSpec((BN, H), lambda i: (i, 0)),
    ],
    out_shape=[
        jax.ShapeDtypeStruct((N, 1), jnp.float32),
        jax.ShapeDtypeStruct((N, H), jnp.float32),
    ],
)


@functools.partial(
    pl.kernel,
    out_type=[
        jax.ShapeDtypeStruct((NC, N, H), jnp.float32),
        jax.ShapeDtypeStruct((NC * RR, CC), jnp.float32),
    ],
    mesh=_mesh,
    compiler_params=_sc_params,
    scratch_types=[
        pltpu.VMEM_SHARED((N, H), jnp.float32),
        pltpu.VMEM_SHARED((RR, CC), jnp.float32),
        pltpu.VMEM((2 * PHCH, ECH), jnp.int32),
        [pltpu.VMEM((ECH, H), jnp.float32)] * 2,
        [pltpu.SemaphoreType.DMA] * 2,
        [pltpu.SemaphoreType.DMA] * 2,
        pltpu.SemaphoreType.DMA,
    ],
)
def _main_kernel(yhat_hbm, norm80_hbm, il2_hbm, S_out, s_out,
                 acc_sh, s_sh, idxb, rows, sem_g, sem_s, sem_i):
    c = lax.axis_index("c")
    s = lax.axis_index("s")
    wid = s * NC + c
    nch = jnp.where(wid < NBIG, 80, 76)
    cbase = 80 * wid - 4 * jnp.maximum(wid - NBIG, 0)

    def idx_stage_start(c0):
        pltpu.async_copy(il2_hbm.at[pl.ds(2 * c0, 2 * PHCH)], idxb, sem_i)

    def idx_stage_wait(c0):
        pltpu.make_async_copy(il2_hbm.at[pl.ds(2 * c0, 2 * PHCH)], idxb,
                              sem_i).wait()

    def gather_start(i, b):
        pltpu.async_copy(yhat_hbm.at[idxb.at[2 * i]], rows[b], sem_g[b])

    def gather_wait(i, b):
        pltpu.make_async_copy(yhat_hbm.at[idxb.at[2 * i]], rows[b],
                              sem_g[b]).wait()

    def scatter_start(i, b):
        pltpu.async_copy(rows[b], acc_sh.at[idxb.at[2 * i + 1]], sem_s[b],
                         add=True)

    def scatter_wait(i, b):
        pltpu.make_async_copy(rows[b], acc_sh.at[idxb.at[2 * i + 1]],
                              sem_s[b]).wait()

    idx_stage_start(cbase)
    off = pl.multiple_of(s * STRIPE, 8)
    pltpu.sync_copy(yhat_hbm.at[pl.ds(off, STRIPE)],
                    acc_sh.at[pl.ds(off, STRIPE)])

    @pl.when(s == 0)
    def _():
        pltpu.sync_copy(yhat_hbm.at[pl.ds(TAIL_OFF, TAIL)],
                        acc_sh.at[pl.ds(TAIL_OFF, TAIL)])

    zeros = jnp.zeros((L,), jnp.float32)

    def zb(i, carry):
        rows[1][i >> 3, pl.ds((i & 7) * L, L)] = zeros
        return carry

    lax.fori_loop(0, (RR // NS) * (CC // L), zb, 0)
    pltpu.sync_copy(rows[1].at[pl.ds(0, RR // NS)],
                    s_sh.at[pl.ds(s * (RR // NS), RR // NS)])
    plsc.subcore_barrier()

    def run_phase(nloc):
        gather_start(0, 0)

        def pair(p, carry):
            for b in range(2):
                i = 2 * p + b
                nb = 1 - b

                @pl.when(i >= 1)
                def _():
                    scatter_wait(i - 1, nb)

                @pl.when(i + 1 < nloc)
                def _():
                    gather_start(i + 1, nb)

                gather_wait(i, b)
                scatter_start(i, b)
            return carry

        lax.fori_loop(0, nloc // 2, pair, 0)
        scatter_wait(nloc - 1, 1)

    idx_stage_wait(cbase)
    run_phase(PHCH)
    pltpu.sync_copy(il2_hbm.at[pl.ds(2 * (cbase + PHCH), 2 * PHCH)], idxb)
    run_phase(nch - PHCH)

    pltpu.sync_copy(norm80_hbm, rows[0].at[pl.ds(0, RR)])
    _zero_2d(rows[1])
    for ph in range(2):
        c0 = cbase + ph * PHCH
        nloc = PHCH if ph == 0 else nch - PHCH
        pltpu.sync_copy(il2_hbm.at[pl.ds(2 * c0, 2 * PHCH)], idxb)

        def sbody(i, carry):
            for g in range(ECH // L):
                d16 = idxb[2 * i + 1, pl.ds(g * L, L)]
                s16 = idxb[2 * i, pl.ds(g * L, L)]
                val = plsc.load_gather(rows[0], [d16 >> 7, d16 & 127])
                plsc.addupdate_scatter(rows[1], [s16 >> 7, s16 & 127], val)
            return carry

        lax.fori_loop(0, nloc, sbody, 0)

    _rowadd_to_shared(rows[1], s_sh)
    plsc.subcore_barrier()
    pltpu.sync_copy(acc_sh.at[pl.ds(off, STRIPE)],
                    S_out.at[c, pl.ds(off, STRIPE)])
    _readout_shared(s_sh, s_out, c, s)

    @pl.when(s == 0)
    def _():
        pltpu.sync_copy(acc_sh.at[pl.ds(TAIL_OFF, TAIL)],
                        S_out.at[c, pl.ds(TAIL_OFF, TAIL)])


def _final_body(S_ref, yhat_ref, norm_ref, sp_ref, b1_ref, w2_ref, b2_ref,
                wp_ref, bp_ref, out_ref, acc):
    i = pl.program_id(0)

    @pl.when(i == 0)
    def _():
        acc[...] = jnp.zeros_like(acc)

    nrm = norm_ref[...]
    ssum = S_ref[0] + S_ref[1] - yhat_ref[...]
    agg = ssum * nrm + b1_ref[...]
    h = jnp.maximum(agg, 0.0)
    stot = jnp.sum(sp_ref[...], axis=1)
    cvec = nrm[:, 0] * (nrm[:, 0] + stot)
    acc[...] += jnp.dot(cvec[None, :], h, preferred_element_type=jnp.float32)

    @pl.when(i == pl.num_programs(0) - 1)
    def _():
        graph = jnp.dot(acc[...] / N, w2_ref[...],
                        preferred_element_type=jnp.float32) + b2_ref[...]
        proj = jnp.dot(graph, wp_ref[...],
                       preferred_element_type=jnp.float32) + bp_ref[...]
        nn = jnp.sqrt(jnp.sum(proj * proj))
        out_ref[...] = proj / jnp.maximum(nn, 1e-12)


_final_call = pl.pallas_call(
    _final_body,
    grid=(GRID,),
    in_specs=[
        pl.BlockSpec((NC, BN, H), lambda i: (0, i, 0)),
        pl.BlockSpec((BN, H), lambda i: (i, 0)),
        pl.BlockSpec((BN, 1), lambda i: (i, 0)),
        pl.BlockSpec((BN, NC), lambda i: (i, 0)),
        pl.BlockSpec((1, H), lambda i: (0, 0)),
        pl.BlockSpec((H, H), lambda i: (0, 0)),
        pl.BlockSpec((1, H), lambda i: (0, 0)),
        pl.BlockSpec((H, H), lambda i: (0, 0)),
        pl.BlockSpec((1, H), lambda i: (0, 0)),
    ],
    out_specs=pl.BlockSpec((1, H), lambda i: (0, 0)),
    out_shape=jax.ShapeDtypeStruct((1, H), jnp.float32),
    scratch_shapes=[pltpu.VMEM((1, H), jnp.float32)],
)


def kernel(x, edge_index, W1, b1, W2, b2, Wp, bp):
    src = edge_index[0]
    dst = edge_index[1]
    y = _mm_call(x, W1)
    deg_lin = _deg_kernel(dst)
    degT = deg_lin.reshape(NC, NP).T
    norm_col, yhat = _scale_call(degT, y)
    il2 = jnp.stack([src.reshape(CHT, ECH), dst.reshape(CHT, ECH)],
                    axis=1).reshape(2 * CHT, ECH)
    il2 = jnp.pad(il2, ((0, 2 * PHCH), (0, 0)))
    norm80 = jnp.pad(norm_col.reshape(N), (0, NP - N)).reshape(RR, CC)
    S_parts, s_lin = _main_kernel(yhat, norm80, il2)
    sT = s_lin.reshape(NC, NP).T
    return _final_call(S_parts, yhat, norm_col, sT,
                       b1.reshape(1, H), W2, b2.reshape(1, H),
                       Wp, bp.reshape(1, H))


# speedup vs baseline: 1.0448x; 1.0448x over previous
import functools

import jax
import jax.numpy as jnp
from jax import lax
from jax.experimental import pallas as pl
from jax.experimental.pallas import tpu as pltpu
from jax.experimental.pallas import tpu_sc as plsc

N = 10000
E = 320000
D = 128
H = 128

NC = 2
NS = 16
L = 16
NW = NC * NS
EPT = E // NW
CHUNK = 80
NCHUNK = EPT // CHUNK
STRIPE = 624
TAIL = N - STRIPE * NS
TAIL_OFF = STRIPE * NS

CC = 128
RR = 80
NP = RR * CC

ECH = 128
CHT = E // ECH
NBIG = 17
PHCH = 40

BN = 1000
GRID = N // BN

_mesh = plsc.VectorSubcoreMesh(core_axis_name="c", subcore_axis_name="s")
_sc_params = pltpu.CompilerParams(needs_layout_passes=False)


def _zero_2d(ref):
    zeros = jnp.zeros((L,), jnp.float32)

    def zbody(i, carry):
        ref[i >> 3, pl.ds((i & 7) * L, L)] = zeros
        return carry

    lax.fori_loop(0, RR * CC // L, zbody, 0)


def _rowadd_to_shared(vref, shref):
    for k in range(RR // L):
        rid = lax.iota(jnp.int32, L) + (k * L)
        pltpu.sync_copy(vref.at[pl.ds(k * L, L)], shref.at[rid], add=True)


def _readout_shared(shref, out_hbm, c, s):
    @pl.when(s < RR // 8)
    def _():
        pltpu.sync_copy(shref.at[pl.ds(s * 8, 8)],
                        out_hbm.at[pl.ds(c * RR + s * 8, 8)])


@functools.partial(
    pl.kernel,
    out_type=jax.ShapeDtypeStruct((NC * RR, CC), jnp.float32),
    mesh=_mesh,
    compiler_params=_sc_params,
    scratch_types=[
        pltpu.VMEM_SHARED((RR, CC), jnp.float32),
        pltpu.VMEM((RR, CC), jnp.float32),
        pltpu.VMEM((EPT,), jnp.int32),
    ],
)
def _deg_kernel(dst_hbm, deg_out, deg_sh, deg_v, dstbuf):
    c = lax.axis_index("c")
    s = lax.axis_index("s")
    wid = s * NC + c
    pltpu.sync_copy(dst_hbm.at[pl.ds(wid * EPT, EPT)], dstbuf)
    _zero_2d(deg_v)
    pltpu.sync_copy(deg_v.at[pl.ds(0, RR // NS)],
                    deg_sh.at[pl.ds(s * (RR // NS), RR // NS)])
    ones = jnp.ones((L,), jnp.float32)

    def body(i, carry):
        idx = dstbuf[pl.ds(i * L, L)]
        plsc.addupdate_scatter(deg_v, [idx >> 7, idx & 127], ones)
        return carry

    lax.fori_loop(0, EPT // L, body, 0)
    plsc.subcore_barrier()
    _rowadd_to_shared(deg_v, deg_sh)
    plsc.subcore_barrier()
    _readout_shared(deg_sh, deg_out, c, s)


def _prep_body(degt_ref, x_ref, w1_ref, norm_ref, yhat_ref):
    deg = jnp.sum(degt_ref[...], axis=1) + 1.0
    nrm = lax.rsqrt(deg)
    norm_ref[...] = nrm[:, None]
    y = jnp.dot(x_ref[...], w1_ref[...], preferred_element_type=jnp.float32)
    yhat_ref[...] = y * nrm[:, None]


_prep_call = pl.pallas_call(
    _prep_body,
    grid=(GRID,),
    in_specs=[
        pl.BlockSpec((BN, NC), lambda i: (i, 0)),
        pl.BlockSpec((BN, D), lambda i: (i, 0)),
        pl.BlockSpec((D, H), lambda i: (0, 0)),
    ],
    out_specs=[
        pl.BlockSpec((BN, 1), lambda i: (i, 0)),
        pl.BlockSpec((BN, H), lambda i: (i, 0)),
    ],
    out_shape=[
        jax.ShapeDtypeStruct((N, 1), jnp.float32),
        jax.ShapeDtypeStruct((N, H), jnp.float32),
    ],
)


@functools.partial(
    pl.kernel,
    out_type=[
        jax.ShapeDtypeStruct((NC, N, H), jnp.float32),
        jax.ShapeDtypeStruct((NC * RR, CC), jnp.float32),
    ],
    mesh=_mesh,
    compiler_params=_sc_params,
    scratch_types=[
        pltpu.VMEM_SHARED((N, H), jnp.float32),
        pltpu.VMEM_SHARED((RR, CC), jnp.float32),
        pltpu.VMEM((2 * PHCH, ECH), jnp.int32),
        [pltpu.VMEM((ECH, H), jnp.float32)] * 2,
        [pltpu.SemaphoreType.DMA] * 2,
        [pltpu.SemaphoreType.DMA] * 2,
        pltpu.SemaphoreType.DMA,
    ],
)
def _main_kernel(yhat_hbm, norm80_hbm, il2_hbm, S_out, s_out,
                 acc_sh, s_sh, idxb, rows, sem_g, sem_s, sem_i):
    c = lax.axis_index("c")
    s = lax.axis_index("s")
    wid = s * NC + c
    nch = jnp.where(wid < NBIG, 80, 76)
    cbase = 80 * wid - 4 * jnp.maximum(wid - NBIG, 0)

    def idx_stage_start(c0):
        pltpu.async_copy(il2_hbm.at[pl.ds(2 * c0, 2 * PHCH)], idxb, sem_i)

    def idx_stage_wait(c0):
        pltpu.make_async_copy(il2_hbm.at[pl.ds(2 * c0, 2 * PHCH)], idxb,
                              sem_i).wait()

    def gather_start(i, b):
        pltpu.async_copy(yhat_hbm.at[idxb.at[2 * i]], rows[b], sem_g[b])

    def gather_wait(i, b):
        pltpu.make_async_copy(yhat_hbm.at[idxb.at[2 * i]], rows[b],
                              sem_g[b]).wait()

    def scatter_start(i, b):
        pltpu.async_copy(rows[b], acc_sh.at[idxb.at[2 * i + 1]], sem_s[b],
                         add=True)

    def scatter_wait(i, b):
        pltpu.make_async_copy(rows[b], acc_sh.at[idxb.at[2 * i + 1]],
                              sem_s[b]).wait()

    idx_stage_start(cbase)
    off = pl.multiple_of(s * STRIPE, 8)
    pltpu.sync_copy(yhat_hbm.at[pl.ds(off, STRIPE)],
                    acc_sh.at[pl.ds(off, STRIPE)])

    @pl.when(s == 0)
    def _():
        pltpu.sync_copy(yhat_hbm.at[pl.ds(TAIL_OFF, TAIL)],
                        acc_sh.at[pl.ds(TAIL_OFF, TAIL)])

    zeros = jnp.zeros((L,), jnp.float32)

    def zb(i, carry):
        rows[1][i >> 3, pl.ds((i & 7) * L, L)] = zeros
        return carry

    lax.fori_loop(0, (RR // NS) * (CC // L), zb, 0)
    pltpu.sync_copy(rows[1].at[pl.ds(0, RR // NS)],
                    s_sh.at[pl.ds(s * (RR // NS), RR // NS)])
    plsc.subcore_barrier()

    def run_phase(nloc):
        gather_start(0, 0)

        def pair(p, carry):
            for b in range(2):
                i = 2 * p + b
                nb = 1 - b

                @pl.when(i >= 1)
                def _():
                    scatter_wait(i - 1, nb)

                @pl.when(i + 1 < nloc)
                def _():
                    gather_start(i + 1, nb)

                gather_wait(i, b)
                scatter_start(i, b)
            return carry

        lax.fori_loop(0, nloc // 2, pair, 0)
        scatter_wait(nloc - 1, 1)

    idx_stage_wait(cbase)
    run_phase(PHCH)
    pltpu.sync_copy(il2_hbm.at[pl.ds(2 * (cbase + PHCH), 2 * PHCH)], idxb)
    run_phase(nch - PHCH)

    pltpu.sync_copy(norm80_hbm, rows[0].at[pl.ds(0, RR)])
    _zero_2d(rows[1])
    for ph in range(2):
        c0 = cbase + ph * PHCH
        nloc = PHCH if ph == 0 else nch - PHCH
        pltpu.sync_copy(il2_hbm.at[pl.ds(2 * c0, 2 * PHCH)], idxb)

        def sbody(i, carry):
            for g in range(ECH // L):
                d16 = idxb[2 * i + 1, pl.ds(g * L, L)]
                s16 = idxb[2 * i, pl.ds(g * L, L)]
                val = plsc.load_gather(rows[0], [d16 >> 7, d16 & 127])
                plsc.addupdate_scatter(rows[1], [s16 >> 7, s16 & 127], val)
            return carry

        lax.fori_loop(0, nloc, sbody, 0)

    _rowadd_to_shared(rows[1], s_sh)
    plsc.subcore_barrier()
    pltpu.sync_copy(acc_sh.at[pl.ds(off, STRIPE)],
                    S_out.at[c, pl.ds(off, STRIPE)])
    _readout_shared(s_sh, s_out, c, s)

    @pl.when(s == 0)
    def _():
        pltpu.sync_copy(acc_sh.at[pl.ds(TAIL_OFF, TAIL)],
                        S_out.at[c, pl.ds(TAIL_OFF, TAIL)])


def _final_body(S_ref, yhat_ref, norm_ref, sp_ref, b1_ref, w2_ref, b2_ref,
                wp_ref, bp_ref, out_ref, acc):
    i = pl.program_id(0)

    @pl.when(i == 0)
    def _():
        acc[...] = jnp.zeros_like(acc)

    nrm = norm_ref[...]
    ssum = S_ref[0] + S_ref[1] - yhat_ref[...]
    agg = ssum * nrm + b1_ref[...]
    h = jnp.maximum(agg, 0.0)
    stot = jnp.sum(sp_ref[...], axis=1)
    cvec = nrm[:, 0] * (nrm[:, 0] + stot)
    acc[...] += jnp.dot(cvec[None, :], h, preferred_element_type=jnp.float32)

    @pl.when(i == pl.num_programs(0) - 1)
    def _():
        graph = jnp.dot(acc[...] / N, w2_ref[...],
                        preferred_element_type=jnp.float32) + b2_ref[...]
        proj = jnp.dot(graph, wp_ref[...],
                       preferred_element_type=jnp.float32) + bp_ref[...]
        nn = jnp.sqrt(jnp.sum(proj * proj))
        out_ref[...] = proj / jnp.maximum(nn, 1e-12)


_final_call = pl.pallas_call(
    _final_body,
    grid=(GRID,),
    in_specs=[
        pl.BlockSpec((NC, BN, H), lambda i: (0, i, 0)),
        pl.BlockSpec((BN, H), lambda i: (i, 0)),
        pl.BlockSpec((BN, 1), lambda i: (i, 0)),
        pl.BlockSpec((BN, NC), lambda i: (i, 0)),
        pl.BlockSpec((1, H), lambda i: (0, 0)),
        pl.BlockSpec((H, H), lambda i: (0, 0)),
        pl.BlockSpec((1, H), lambda i: (0, 0)),
        pl.BlockSpec((H, H), lambda i: (0, 0)),
        pl.BlockSpec((1, H), lambda i: (0, 0)),
    ],
    out_specs=pl.BlockSpec((1, H), lambda i: (0, 0)),
    out_shape=jax.ShapeDtypeStruct((1, H), jnp.float32),
    scratch_shapes=[pltpu.VMEM((1, H), jnp.float32)],
)


def kernel(x, edge_index, W1, b1, W2, b2, Wp, bp):
    src = edge_index[0]
    dst = edge_index[1]
    deg_lin = _deg_kernel(dst)
    degT = deg_lin.reshape(NC, NP).T
    norm_col, yhat = _prep_call(degT, x, W1)
    il2 = jnp.stack([src.reshape(CHT, ECH), dst.reshape(CHT, ECH)],
                    axis=1).reshape(2 * CHT, ECH)
    il2 = jnp.pad(il2, ((0, 2 * PHCH), (0, 0)))
    norm80 = jnp.pad(norm_col.reshape(N), (0, NP - N)).reshape(RR, CC)
    S_parts, s_lin = _main_kernel(yhat, norm80, il2)
    sT = s_lin.reshape(NC, NP).T
    return _final_call(S_parts, yhat, norm_col, sT,
                       b1.reshape(1, H), W2, b2.reshape(1, H),
                       Wp, bp.reshape(1, H))


# speedup vs baseline: 1.0708x; 1.0248x over previous
import functools

import jax
import jax.numpy as jnp
from jax import lax
from jax.experimental import pallas as pl
from jax.experimental.pallas import tpu as pltpu
from jax.experimental.pallas import tpu_sc as plsc

N = 10000
E = 320000
D = 128
H = 128

NC = 2
NS = 16
L = 16
NW = NC * NS
EPT = E // NW
STRIPE = 624
TAIL = N - STRIPE * NS
TAIL_OFF = STRIPE * NS

CC = 128
RR = 80
NP = RR * CC

ECH = 128
CHT = E // ECH
NBIG = 17
PHCH = 40

BN = 2000
GRID = N // BN

_mesh = plsc.VectorSubcoreMesh(core_axis_name="c", subcore_axis_name="s")
_sc_params = pltpu.CompilerParams(needs_layout_passes=False)


def _zero_2d(ref):
    zeros = jnp.zeros((L,), jnp.float32)

    def zbody(i, carry):
        ref[i >> 3, pl.ds((i & 7) * L, L)] = zeros
        return carry

    lax.fori_loop(0, RR * CC // L, zbody, 0)


def _rowadd_to_shared(vref, shref):
    for k in range(RR // L):
        rid = lax.iota(jnp.int32, L) + (k * L)
        pltpu.sync_copy(vref.at[pl.ds(k * L, L)], shref.at[rid], add=True)


def _readout_shared(shref, out_hbm, c, s):
    @pl.when(s < RR // 8)
    def _():
        pltpu.sync_copy(shref.at[pl.ds(s * 8, 8)],
                        out_hbm.at[pl.ds(c * RR + s * 8, 8)])


@functools.partial(
    pl.kernel,
    out_type=jax.ShapeDtypeStruct((NC * RR, CC), jnp.float32),
    mesh=_mesh,
    compiler_params=_sc_params,
    scratch_types=[
        pltpu.VMEM_SHARED((RR, CC), jnp.float32),
        pltpu.VMEM((RR, CC), jnp.float32),
        pltpu.VMEM((EPT,), jnp.int32),
    ],
)
def _deg_kernel(dst_hbm, deg_out, deg_sh, deg_v, dstbuf):
    c = lax.axis_index("c")
    s = lax.axis_index("s")
    wid = s * NC + c
    pltpu.sync_copy(dst_hbm.at[pl.ds(wid * EPT, EPT)], dstbuf)
    _zero_2d(deg_v)
    pltpu.sync_copy(deg_v.at[pl.ds(0, RR // NS)],
                    deg_sh.at[pl.ds(s * (RR // NS), RR // NS)])
    ones = jnp.ones((L,), jnp.float32)

    def body(i, carry):
        idx = dstbuf[pl.ds(i * L, L)]
        plsc.addupdate_scatter(deg_v, [idx >> 7, idx & 127], ones)
        return carry

    lax.fori_loop(0, EPT // L, body, 0)
    plsc.subcore_barrier()
    _rowadd_to_shared(deg_v, deg_sh)
    plsc.subcore_barrier()
    _readout_shared(deg_sh, deg_out, c, s)


def _prep_body(degt_ref, x_ref, w1_ref, norm_ref, yhat_ref):
    deg = jnp.sum(degt_ref[...], axis=1) + 1.0
    nrm = lax.rsqrt(deg)
    norm_ref[...] = nrm[:, None]
    y = jnp.dot(x_ref[...], w1_ref[...], preferred_element_type=jnp.float32)
    yhat_ref[...] = y * nrm[:, None]


_prep_call = pl.pallas_call(
    _prep_body,
    grid=(GRID,),
    in_specs=[
        pl.BlockSpec((BN, NC), lambda i: (i, 0)),
        pl.BlockSpec((BN, D), lambda i: (i, 0)),
        pl.BlockSpec((D, H), lambda i: (0, 0)),
    ],
    out_specs=[
        pl.BlockSpec((BN, 1), lambda i: (i, 0)),
        pl.BlockSpec((BN, H), lambda i: (i, 0)),
    ],
    out_shape=[
        jax.ShapeDtypeStruct((N, 1), jnp.float32),
        jax.ShapeDtypeStruct((N, H), jnp.float32),
    ],
)


@functools.partial(
    pl.kernel,
    out_type=[
        jax.ShapeDtypeStruct((NC, N, H), jnp.float32),
        jax.ShapeDtypeStruct((NC * RR, CC), jnp.float32),
    ],
    mesh=_mesh,
    compiler_params=_sc_params,
    scratch_types=[
        pltpu.VMEM_SHARED((N, H), jnp.float32),
        pltpu.VMEM_SHARED((RR, CC), jnp.float32),
        pltpu.VMEM((2 * PHCH, ECH), jnp.int32),
        [pltpu.VMEM((ECH, H), jnp.float32)] * 2,
        [pltpu.SemaphoreType.DMA] * 2,
        [pltpu.SemaphoreType.DMA] * 2,
        pltpu.SemaphoreType.DMA,
    ],
)
def _main_kernel(yhat_hbm, norm80_hbm, il2_hbm, S_out, s_out,
                 acc_sh, s_sh, idxb, rows, sem_g, sem_s, sem_i):
    c = lax.axis_index("c")
    s = lax.axis_index("s")
    wid = s * NC + c
    nch = jnp.where(wid < NBIG, 80, 76)
    cbase = 80 * wid - 4 * jnp.maximum(wid - NBIG, 0)

    def idx_stage_start(c0):
        pltpu.async_copy(il2_hbm.at[pl.ds(2 * c0, 2 * PHCH)], idxb, sem_i)

    def idx_stage_wait(c0):
        pltpu.make_async_copy(il2_hbm.at[pl.ds(2 * c0, 2 * PHCH)], idxb,
                              sem_i).wait()

    def gather_start(i, b):
        pltpu.async_copy(yhat_hbm.at[idxb.at[2 * i]], rows[b], sem_g[b])

    def gather_wait(i, b):
        pltpu.make_async_copy(yhat_hbm.at[idxb.at[2 * i]], rows[b],
                              sem_g[b]).wait()

    def scatter_start(i, b):
        pltpu.async_copy(rows[b], acc_sh.at[idxb.at[2 * i + 1]], sem_s[b],
                         add=True)

    def scatter_wait(i, b):
        pltpu.make_async_copy(rows[b], acc_sh.at[idxb.at[2 * i + 1]],
                              sem_s[b]).wait()

    idx_stage_start(cbase)
    off = pl.multiple_of(s * STRIPE, 8)
    pltpu.sync_copy(yhat_hbm.at[pl.ds(off, STRIPE)],
                    acc_sh.at[pl.ds(off, STRIPE)])

    @pl.when(s == 0)
    def _():
        pltpu.sync_copy(yhat_hbm.at[pl.ds(TAIL_OFF, TAIL)],
                        acc_sh.at[pl.ds(TAIL_OFF, TAIL)])

    zeros = jnp.zeros((L,), jnp.float32)

    def zb(i, carry):
        rows[1][i >> 3, pl.ds((i & 7) * L, L)] = zeros
        return carry

    lax.fori_loop(0, (RR // NS) * (CC // L), zb, 0)
    pltpu.sync_copy(rows[1].at[pl.ds(0, RR // NS)],
                    s_sh.at[pl.ds(s * (RR // NS), RR // NS)])
    plsc.subcore_barrier()

    def run_phase(nloc):
        gather_start(0, 0)

        def pair(p, carry):
            for b in range(2):
                i = 2 * p + b
                nb = 1 - b

                @pl.when(i >= 1)
                def _():
                    scatter_wait(i - 1, nb)

                @pl.when(i + 1 < nloc)
                def _():
                    gather_start(i + 1, nb)

                gather_wait(i, b)
                scatter_start(i, b)
            return carry

        lax.fori_loop(0, nloc // 2, pair, 0)
        scatter_wait(nloc - 1, 1)

    idx_stage_wait(cbase)
    run_phase(PHCH)
    pltpu.sync_copy(il2_hbm.at[pl.ds(2 * (cbase + PHCH), 2 * PHCH)], idxb)
    run_phase(nch - PHCH)

    pltpu.sync_copy(norm80_hbm, rows[0].at[pl.ds(0, RR)])
    _zero_2d(rows[1])
    for ph in range(2):
        c0 = cbase + ph * PHCH
        nloc = PHCH if ph == 0 else nch - PHCH
        pltpu.sync_copy(il2_hbm.at[pl.ds(2 * c0, 2 * PHCH)], idxb)

        def sbody(i, carry):
            for g in range(ECH // L):
                d16 = idxb[2 * i + 1, pl.ds(g * L, L)]
                s16 = idxb[2 * i, pl.ds(g * L, L)]
                val = plsc.load_gather(rows[0], [d16 >> 7, d16 & 127])
                plsc.addupdate_scatter(rows[1], [s16 >> 7, s16 & 127], val)
            return carry

        lax.fori_loop(0, nloc, sbody, 0)

    _rowadd_to_shared(rows[1], s_sh)
    plsc.subcore_barrier()
    pltpu.sync_copy(acc_sh.at[pl.ds(off, STRIPE)],
                    S_out.at[c, pl.ds(off, STRIPE)])
    _readout_shared(s_sh, s_out, c, s)

    @pl.when(s == 0)
    def _():
        pltpu.sync_copy(acc_sh.at[pl.ds(TAIL_OFF, TAIL)],
                        S_out.at[c, pl.ds(TAIL_OFF, TAIL)])


def _final_body(S_ref, yhat_ref, norm_ref, sp_ref, b1_ref, w2_ref, b2_ref,
                wp_ref, bp_ref, out_ref, acc):
    i = pl.program_id(0)

    @pl.when(i == 0)
    def _():
        acc[...] = jnp.zeros_like(acc)

    nrm = norm_ref[...]
    ssum = S_ref[0] + S_ref[1] - yhat_ref[...]
    agg = ssum * nrm + b1_ref[...]
    h = jnp.maximum(agg, 0.0)
    stot = jnp.sum(sp_ref[...], axis=1)
    cvec = nrm[:, 0] * (nrm[:, 0] + stot)
    acc[...] += jnp.dot(cvec[None, :], h, preferred_element_type=jnp.float32)

    @pl.when(i == pl.num_programs(0) - 1)
    def _():
        graph = jnp.dot(acc[...] / N, w2_ref[...],
                        preferred_element_type=jnp.float32) + b2_ref[...]
        proj = jnp.dot(graph, wp_ref[...],
                       preferred_element_type=jnp.float32) + bp_ref[...]
        nn = jnp.sqrt(jnp.sum(proj * proj))
        out_ref[...] = proj / jnp.maximum(nn, 1e-12)


_final_call = pl.pallas_call(
    _final_body,
    grid=(GRID,),
    in_specs=[
        pl.BlockSpec((NC, BN, H), lambda i: (0, i, 0)),
        pl.BlockSpec((BN, H), lambda i: (i, 0)),
        pl.BlockSpec((BN, 1), lambda i: (i, 0)),
        pl.BlockSpec((BN, NC), lambda i: (i, 0)),
        pl.BlockSpec((1, H), lambda i: (0, 0)),
        pl.BlockSpec((H, H), lambda i: (0, 0)),
        pl.BlockSpec((1, H), lambda i: (0, 0)),
        pl.BlockSpec((H, H), lambda i: (0, 0)),
        pl.BlockSpec((1, H), lambda i: (0, 0)),
    ],
    out_specs=pl.BlockSpec((1, H), lambda i: (0, 0)),
    out_shape=jax.ShapeDtypeStruct((1, H), jnp.float32),
    scratch_shapes=[pltpu.VMEM((1, H), jnp.float32)],
)


def kernel(x, edge_index, W1, b1, W2, b2, Wp, bp):
    src = edge_index[0]
    dst = edge_index[1]
    deg_lin = _deg_kernel(dst)
    degT = deg_lin.reshape(NC, NP).T
    norm_col, yhat = _prep_call(degT, x, W1)
    il2 = jnp.stack([src.reshape(CHT, ECH), dst.reshape(CHT, ECH)],
                    axis=1).reshape(2 * CHT, ECH)
    il2 = jnp.pad(il2, ((0, 2 * PHCH), (0, 0)))
    norm80 = jnp.pad(norm_col.reshape(N), (0, NP - N)).reshape(RR, CC)
    S_parts, s_lin = _main_kernel(yhat, norm80, il2)
    sT = s_lin.reshape(NC, NP).T
    return _final_call(S_parts, yhat, norm_col, sT,
                       b1.reshape(1, H), W2, b2.reshape(1, H),
                       Wp, bp.reshape(1, H))


# speedup vs baseline: 1.0827x; 1.0111x over previous
import functools

import jax
import jax.numpy as jnp
from jax import lax
from jax.experimental import pallas as pl
from jax.experimental.pallas import tpu as pltpu
from jax.experimental.pallas import tpu_sc as plsc

N = 10000
E = 320000
D = 128
H = 128

NC = 2
NS = 16
L = 16
NW = NC * NS
EPT = E // NW
STRIPE = 624
TAIL = N - STRIPE * NS
TAIL_OFF = STRIPE * NS

CC = 128
RR = 80
NP = RR * CC

ECH = 128
CHT = E // ECH
NBIG = 17
PHCH = 40

BN = 5000
GRID = N // BN

_mesh = plsc.VectorSubcoreMesh(core_axis_name="c", subcore_axis_name="s")
_sc_params = pltpu.CompilerParams(needs_layout_passes=False)


def _zero_2d(ref):
    zeros = jnp.zeros((L,), jnp.float32)

    def zbody(i, carry):
        ref[i >> 3, pl.ds((i & 7) * L, L)] = zeros
        return carry

    lax.fori_loop(0, RR * CC // L, zbody, 0)


def _rowadd_to_shared(vref, shref):
    for k in range(RR // L):
        rid = lax.iota(jnp.int32, L) + (k * L)
        pltpu.sync_copy(vref.at[pl.ds(k * L, L)], shref.at[rid], add=True)


def _readout_shared(shref, out_hbm, c, s):
    @pl.when(s < RR // 8)
    def _():
        pltpu.sync_copy(shref.at[pl.ds(s * 8, 8)],
                        out_hbm.at[pl.ds(c * RR + s * 8, 8)])


@functools.partial(
    pl.kernel,
    out_type=jax.ShapeDtypeStruct((NC * RR, CC), jnp.float32),
    mesh=_mesh,
    compiler_params=_sc_params,
    scratch_types=[
        pltpu.VMEM_SHARED((RR, CC), jnp.float32),
        pltpu.VMEM((RR, CC), jnp.float32),
        pltpu.VMEM((EPT,), jnp.int32),
    ],
)
def _deg_kernel(dst_hbm, deg_out, deg_sh, deg_v, dstbuf):
    c = lax.axis_index("c")
    s = lax.axis_index("s")
    wid = s * NC + c
    pltpu.sync_copy(dst_hbm.at[pl.ds(wid * EPT, EPT)], dstbuf)
    _zero_2d(deg_v)
    pltpu.sync_copy(deg_v.at[pl.ds(0, RR // NS)],
                    deg_sh.at[pl.ds(s * (RR // NS), RR // NS)])
    ones = jnp.ones((L,), jnp.float32)

    def body(i, carry):
        idx = dstbuf[pl.ds(i * L, L)]
        plsc.addupdate_scatter(deg_v, [idx >> 7, idx & 127], ones)
        return carry

    lax.fori_loop(0, EPT // L, body, 0)
    plsc.subcore_barrier()
    _rowadd_to_shared(deg_v, deg_sh)
    plsc.subcore_barrier()
    _readout_shared(deg_sh, deg_out, c, s)


def _prep_body(degt_ref, x_ref, w1_ref, norm_ref, yhat_ref):
    deg = jnp.sum(degt_ref[...], axis=1) + 1.0
    nrm = lax.rsqrt(deg)
    norm_ref[...] = nrm[:, None]
    y = jnp.dot(x_ref[...], w1_ref[...], preferred_element_type=jnp.float32)
    yhat_ref[...] = y * nrm[:, None]


_prep_call = pl.pallas_call(
    _prep_body,
    grid=(GRID,),
    in_specs=[
        pl.BlockSpec((BN, NC), lambda i: (i, 0)),
        pl.BlockSpec((BN, D), lambda i: (i, 0)),
        pl.BlockSpec((D, H), lambda i: (0, 0)),
    ],
    out_specs=[
        pl.BlockSpec((BN, 1), lambda i: (i, 0)),
        pl.BlockSpec((BN, H), lambda i: (i, 0)),
    ],
    out_shape=[
        jax.ShapeDtypeStruct((N, 1), jnp.float32),
        jax.ShapeDtypeStruct((N, H), jnp.float32),
    ],
)


@functools.partial(
    pl.kernel,
    out_type=[
        jax.ShapeDtypeStruct((NC, N, H), jnp.float32),
        jax.ShapeDtypeStruct((NC * RR, CC), jnp.float32),
    ],
    mesh=_mesh,
    compiler_params=_sc_params,
    scratch_types=[
        pltpu.VMEM_SHARED((N, H), jnp.float32),
        pltpu.VMEM_SHARED((RR, CC), jnp.float32),
        pltpu.VMEM((2 * PHCH, ECH), jnp.int32),
        [pltpu.VMEM((ECH, H), jnp.float32)] * 2,
        [pltpu.SemaphoreType.DMA] * 2,
        [pltpu.SemaphoreType.DMA] * 2,
        pltpu.SemaphoreType.DMA,
    ],
)
def _main_kernel(yhat_hbm, norm80_hbm, il2_hbm, S_out, s_out,
                 acc_sh, s_sh, idxb, rows, sem_g, sem_s, sem_i):
    c = lax.axis_index("c")
    s = lax.axis_index("s")
    wid = s * NC + c
    nch = jnp.where(wid < NBIG, 80, 76)
    cbase = 80 * wid - 4 * jnp.maximum(wid - NBIG, 0)

    def idx_stage_start(c0):
        pltpu.async_copy(il2_hbm.at[pl.ds(2 * c0, 2 * PHCH)], idxb, sem_i)

    def idx_stage_wait(c0):
        pltpu.make_async_copy(il2_hbm.at[pl.ds(2 * c0, 2 * PHCH)], idxb,
                              sem_i).wait()

    def gather_start(i, b):
        pltpu.async_copy(yhat_hbm.at[idxb.at[2 * i]], rows[b], sem_g[b])

    def gather_wait(i, b):
        pltpu.make_async_copy(yhat_hbm.at[idxb.at[2 * i]], rows[b],
                              sem_g[b]).wait()

    def scatter_start(i, b):
        pltpu.async_copy(rows[b], acc_sh.at[idxb.at[2 * i + 1]], sem_s[b],
                         add=True)

    def scatter_wait(i, b):
        pltpu.make_async_copy(rows[b], acc_sh.at[idxb.at[2 * i + 1]],
                              sem_s[b]).wait()

    idx_stage_start(cbase)
    off = pl.multiple_of(s * STRIPE, 8)
    pltpu.sync_copy(yhat_hbm.at[pl.ds(off, STRIPE)],
                    acc_sh.at[pl.ds(off, STRIPE)])

    @pl.when(s == 0)
    def _():
        pltpu.sync_copy(yhat_hbm.at[pl.ds(TAIL_OFF, TAIL)],
                        acc_sh.at[pl.ds(TAIL_OFF, TAIL)])

    zeros = jnp.zeros((L,), jnp.float32)

    def zb(i, carry):
        rows[1][i >> 3, pl.ds((i & 7) * L, L)] = zeros
        return carry

    lax.fori_loop(0, (RR // NS) * (CC // L), zb, 0)
    pltpu.sync_copy(rows[1].at[pl.ds(0, RR // NS)],
                    s_sh.at[pl.ds(s * (RR // NS), RR // NS)])
    plsc.subcore_barrier()

    def run_phase(nloc):
        gather_start(0, 0)

        def pair(p, carry):
            for b in range(2):
                i = 2 * p + b
                nb = 1 - b

                @pl.when(i >= 1)
                def _():
                    scatter_wait(i - 1, nb)

                @pl.when(i + 1 < nloc)
                def _():
                    gather_start(i + 1, nb)

                gather_wait(i, b)
                scatter_start(i, b)
            return carry

        lax.fori_loop(0, nloc // 2, pair, 0)
        scatter_wait(nloc - 1, 1)

    idx_stage_wait(cbase)
    run_phase(PHCH)
    pltpu.sync_copy(il2_hbm.at[pl.ds(2 * (cbase + PHCH), 2 * PHCH)], idxb)
    run_phase(nch - PHCH)

    pltpu.sync_copy(norm80_hbm, rows[0].at[pl.ds(0, RR)])
    _zero_2d(rows[1])
    for ph in range(2):
        c0 = cbase + ph * PHCH
        nloc = PHCH if ph == 0 else nch - PHCH
        pltpu.sync_copy(il2_hbm.at[pl.ds(2 * c0, 2 * PHCH)], idxb)

        def sbody(i, carry):
            for g in range(ECH // L):
                d16 = idxb[2 * i + 1, pl.ds(g * L, L)]
                s16 = idxb[2 * i, pl.ds(g * L, L)]
                val = plsc.load_gather(rows[0], [d16 >> 7, d16 & 127])
                plsc.addupdate_scatter(rows[1], [s16 >> 7, s16 & 127], val)
            return carry

        lax.fori_loop(0, nloc, sbody, 0)

    _rowadd_to_shared(rows[1], s_sh)
    plsc.subcore_barrier()
    pltpu.sync_copy(acc_sh.at[pl.ds(off, STRIPE)],
                    S_out.at[c, pl.ds(off, STRIPE)])
    _readout_shared(s_sh, s_out, c, s)

    @pl.when(s == 0)
    def _():
        pltpu.sync_copy(acc_sh.at[pl.ds(TAIL_OFF, TAIL)],
                        S_out.at[c, pl.ds(TAIL_OFF, TAIL)])


def _final_body(S_ref, yhat_ref, norm_ref, sp_ref, b1_ref, w2_ref, b2_ref,
                wp_ref, bp_ref, out_ref, acc):
    i = pl.program_id(0)

    @pl.when(i == 0)
    def _():
        acc[...] = jnp.zeros_like(acc)

    nrm = norm_ref[...]
    ssum = S_ref[0] + S_ref[1] - yhat_ref[...]
    agg = ssum * nrm + b1_ref[...]
    h = jnp.maximum(agg, 0.0)
    stot = jnp.sum(sp_ref[...], axis=1)
    cvec = nrm[:, 0] * (nrm[:, 0] + stot)
    acc[...] += jnp.dot(cvec[None, :], h, preferred_element_type=jnp.float32)

    @pl.when(i == pl.num_programs(0) - 1)
    def _():
        graph = jnp.dot(acc[...] / N, w2_ref[...],
                        preferred_element_type=jnp.float32) + b2_ref[...]
        proj = jnp.dot(graph, wp_ref[...],
                       preferred_element_type=jnp.float32) + bp_ref[...]
        nn = jnp.sqrt(jnp.sum(proj * proj))
        out_ref[...] = proj / jnp.maximum(nn, 1e-12)


_final_call = pl.pallas_call(
    _final_body,
    grid=(GRID,),
    in_specs=[
        pl.BlockSpec((NC, BN, H), lambda i: (0, i, 0)),
        pl.BlockSpec((BN, H), lambda i: (i, 0)),
        pl.BlockSpec((BN, 1), lambda i: (i, 0)),
        pl.BlockSpec((BN, NC), lambda i: (i, 0)),
        pl.BlockSpec((1, H), lambda i: (0, 0)),
        pl.BlockSpec((H, H), lambda i: (0, 0)),
        pl.BlockSpec((1, H), lambda i: (0, 0)),
        pl.BlockSpec((H, H), lambda i: (0, 0)),
        pl.BlockSpec((1, H), lambda i: (0, 0)),
    ],
    out_specs=pl.BlockSpec((1, H), lambda i: (0, 0)),
    out_shape=jax.ShapeDtypeStruct((1, H), jnp.float32),
    scratch_shapes=[pltpu.VMEM((1, H), jnp.float32)],
)


def kernel(x, edge_index, W1, b1, W2, b2, Wp, bp):
    src = edge_index[0]
    dst = edge_index[1]
    deg_lin = _deg_kernel(dst)
    degT = deg_lin.reshape(NC, NP).T
    norm_col, yhat = _prep_call(degT, x, W1)
    il2 = jnp.stack([src.reshape(CHT, ECH), dst.reshape(CHT, ECH)],
                    axis=1).reshape(2 * CHT, ECH)
    il2 = jnp.pad(il2, ((0, 2 * PHCH), (0, 0)))
    norm80 = jnp.pad(norm_col.reshape(N), (0, NP - N)).reshape(RR, CC)
    S_parts, s_lin = _main_kernel(yhat, norm80, il2)
    sT = s_lin.reshape(NC, NP).T
    return _final_call(S_parts, yhat, norm_col, sT,
                       b1.reshape(1, H), W2, b2.reshape(1, H),
                       Wp, bp.reshape(1, H))
